# baseline (speedup 1.0000x reference)
import functools

import jax
import jax.numpy as jnp
from jax import lax
from jax.experimental import pallas as pl
from jax.experimental.pallas import tpu as pltpu
from jax.experimental.pallas import tpu_sc as plsc

B, N, D = 1024, 128, 128
RC = 64

NC, NS = 2, 16
NW = NC * NS
BPW = B // NW
ZWORDS = N * D
L = 16


def _rank_body(rand_ref, row_id_ref):
    i = pl.program_id(0)
    r = rand_ref[...]
    rj = r[:, None, :]
    rk = r[:, :, None]
    k_iota = lax.broadcasted_iota(jnp.int32, (RC, N, N), 1)
    j_iota = lax.broadcasted_iota(jnp.int32, (RC, N, N), 2)
    lt = (rk < rj) | ((rk == rj) & (k_iota < j_iota))
    ranks = jnp.sum(lt.astype(jnp.int32), axis=1)
    bidx = i * RC + lax.broadcasted_iota(jnp.int32, (RC, N), 0)
    row_id_ref[...] = bidx * N + ranks


def _sc_body(idx_hbm, ident_hbm, row_hbm, col_hbm, zbuf, ibuf, idx_v, zsem, ssem):
    wid = lax.axis_index("s") * NC + lax.axis_index("c")
    base = wid * BPW * ZWORDS

    def zstep(i, carry):
        zbuf[pl.ds(i * L, L)] = jnp.zeros((L,), jnp.float32)
        return carry

    lax.fori_loop(0, ZWORDS // L, zstep, 0)
    pltpu.sync_copy(ident_hbm, ibuf)
    pltpu.sync_copy(idx_hbm.at[wid], idx_v)

    copies = []
    for t in range(BPW):
        copies.append(
            pltpu.async_copy(zbuf, row_hbm.at[pl.ds(base + t * ZWORDS, ZWORDS)], zsem)
        )
        copies.append(pltpu.async_copy(ibuf, col_hbm.at[idx_v.at[t]], ssem))
    for cp in copies:
        cp.wait()


_sc_both = functools.partial(
    pl.kernel,
    out_type=(
        jax.ShapeDtypeStruct((B * N * D,), jnp.float32),
        jax.ShapeDtypeStruct((B * N, D), jnp.float32),
    ),
    mesh=plsc.VectorSubcoreMesh(core_axis_name="c", subcore_axis_name="s"),
    scratch_types=[
        pltpu.VMEM((ZWORDS,), jnp.float32),
        pltpu.VMEM((N, D), jnp.float32),
        pltpu.VMEM((BPW, N), jnp.int32),
        pltpu.SemaphoreType.DMA,
        pltpu.SemaphoreType.DMA,
    ],
)(_sc_body)


def kernel(distance_matrix):
    rand = jax.random.uniform(jax.random.key(42), (B, N), dtype=jnp.float32)
    row_ids = pl.pallas_call(
        _rank_body,
        grid=(B // RC,),
        in_specs=[pl.BlockSpec((RC, N), lambda i: (i, 0))],
        out_specs=pl.BlockSpec((RC, N), lambda i: (i, 0)),
        out_shape=jax.ShapeDtypeStruct((B, N), jnp.int32),
    )(rand)
    ident = jnp.eye(N, dtype=jnp.float32)
    row_flat, col_flat = _sc_both(row_ids.reshape(NW, BPW, N), ident)
    return (row_flat.reshape(B, N, D), col_flat.reshape(B, N, D), distance_matrix)


# speedup vs baseline: 1.2692x; 1.2692x over previous
import functools

import jax
import jax.numpy as jnp
from jax import lax
from jax.experimental import pallas as pl
from jax.experimental.pallas import tpu as pltpu
from jax.experimental.pallas import tpu_sc as plsc

B, N, D = 1024, 128, 128
BC = 32

NC, NS = 2, 16
NW = NC * NS
BPW = B // NW
ZWORDS = N * D
L = 16


def _col_body(rand_ref, col_ref):
    r = rand_ref[...]
    rj = r[:, None, :]
    rk = r[:, :, None]
    lt = rk < rj
    ranks = jnp.sum(lt.astype(jnp.int32), axis=1)
    n_iota = lax.broadcasted_iota(jnp.int32, (BC, N, N), 1)
    col_ref[...] = (ranks[:, None, :] == n_iota).astype(jnp.float32)


def _sc_row_body(out_hbm, zbuf, sem):
    wid = lax.axis_index("s") * NC + lax.axis_index("c")
    base = wid * BPW * ZWORDS

    def zstep(i, carry):
        zbuf[pl.ds(i * L, L)] = jnp.zeros((L,), jnp.float32)
        return carry

    lax.fori_loop(0, ZWORDS // L, zstep, 0)
    copies = [
        pltpu.async_copy(zbuf, out_hbm.at[pl.ds(base + t * ZWORDS, ZWORDS)], sem)
        for t in range(BPW)
    ]
    for cp in copies:
        cp.wait()


_sc_row = functools.partial(
    pl.kernel,
    out_type=jax.ShapeDtypeStruct((B * N * D,), jnp.float32),
    mesh=plsc.VectorSubcoreMesh(core_axis_name="c", subcore_axis_name="s"),
    scratch_types=[
        pltpu.VMEM((ZWORDS,), jnp.float32),
        pltpu.SemaphoreType.DMA,
    ],
)(_sc_row_body)


def kernel(distance_matrix):
    rand = jax.random.uniform(jax.random.key(42), (B, N), dtype=jnp.float32)
    row_flat = _sc_row()
    col_emb = pl.pallas_call(
        _col_body,
        grid=(B // BC,),
        in_specs=[pl.BlockSpec((BC, N), lambda i: (i, 0))],
        out_specs=pl.BlockSpec((BC, N, D), lambda i: (i, 0, 0)),
        out_shape=jax.ShapeDtypeStruct((B, N, D), jnp.float32),
    )(rand)
    return (row_flat.reshape(B, N, D), col_emb, distance_matrix)


# speedup vs baseline: 1.3152x; 1.0362x over previous
import functools

import jax
import jax.numpy as jnp
from jax import lax
from jax.experimental import pallas as pl
from jax.experimental.pallas import tpu as pltpu
from jax.experimental.pallas import tpu_sc as plsc

B, N, D = 1024, 128, 128
BC = 64

NC, NS = 2, 16
NW = NC * NS
BPW = B // NW
ZWORDS = N * D
L = 16


def _col_body(rand_ref, col_ref):
    r = rand_ref[...]
    rj = r[:, None, :]
    rk = r[:, :, None]
    lt = rk < rj
    ranks = jnp.sum(lt.astype(jnp.int32), axis=1)
    n_iota = lax.broadcasted_iota(jnp.int32, (BC, N, N), 1)
    col_ref[...] = (ranks[:, None, :] == n_iota).astype(jnp.float32)


def _sc_row_body(out_hbm, zbuf, sem):
    wid = lax.axis_index("s") * NC + lax.axis_index("c")
    base = wid * BPW * ZWORDS

    def zstep(i, carry):
        zbuf[pl.ds(i * L, L)] = jnp.zeros((L,), jnp.float32)
        return carry

    lax.fori_loop(0, ZWORDS // L, zstep, 0)
    copies = [
        pltpu.async_copy(zbuf, out_hbm.at[pl.ds(base + t * ZWORDS, ZWORDS)], sem)
        for t in range(BPW)
    ]
    for cp in copies:
        cp.wait()


_sc_row = functools.partial(
    pl.kernel,
    out_type=jax.ShapeDtypeStruct((B * N * D,), jnp.float32),
    mesh=plsc.VectorSubcoreMesh(core_axis_name="c", subcore_axis_name="s"),
    scratch_types=[
        pltpu.VMEM((ZWORDS,), jnp.float32),
        pltpu.SemaphoreType.DMA,
    ],
)(_sc_row_body)


def kernel(distance_matrix):
    rand = jax.random.uniform(jax.random.key(42), (B, N), dtype=jnp.float32)
    row_flat = _sc_row()
    col_emb = pl.pallas_call(
        _col_body,
        grid=(B // BC,),
        in_specs=[pl.BlockSpec((BC, N), lambda i: (i, 0))],
        out_specs=pl.BlockSpec((BC, N, D), lambda i: (i, 0, 0)),
        out_shape=jax.ShapeDtypeStruct((B, N, D), jnp.float32),
    )(rand)
    return (row_flat.reshape(B, N, D), col_emb, distance_matrix)
